# baseline (speedup 1.0000x reference)
import functools

import jax
import jax.numpy as jnp
from jax import lax
from jax.experimental import pallas as pl
from jax.experimental.pallas import tpu as pltpu
from jax.experimental.pallas import tpu_sc as plsc

N_NODES = 10000
D_FEAT = 128
E_POS = 160000
E_TOTAL = 320000

NC = 2
NS = 16
NW = NC * NS
EPW = E_TOTAL // NW
CREAL = 125
CHUNK = 128
NCHUNK = EPW // CREAL


def _make_sc_logits():
    mesh = plsc.VectorSubcoreMesh(
        core_axis_name="c", subcore_axis_name="s", num_cores=NC, num_subcores=NS
    )

    @functools.partial(
        pl.kernel,
        mesh=mesh,
        compiler_params=pltpu.CompilerParams(
            needs_layout_passes=False, use_tc_tiling_on_sc=False),
        out_type=jax.ShapeDtypeStruct((NW, NCHUNK, CHUNK), jnp.float32),
        scratch_types=[
            pltpu.VMEM((NCHUNK, CHUNK), jnp.int32),
            pltpu.VMEM((NCHUNK, CHUNK), jnp.int32),
            pltpu.VMEM((CHUNK, D_FEAT), jnp.bfloat16),
            pltpu.VMEM((CHUNK, D_FEAT), jnp.bfloat16),
            pltpu.VMEM((CHUNK, D_FEAT), jnp.bfloat16),
            pltpu.VMEM((CHUNK, D_FEAT), jnp.bfloat16),
            pltpu.VMEM((NCHUNK, CHUNK), jnp.float32),
            pltpu.VMEM_SHARED((N_NODES, D_FEAT), jnp.bfloat16),
            pltpu.SemaphoreType.DMA,
            pltpu.SemaphoreType.DMA,
        ],
    )
    def sc_logits(z_hbm, src_hbm, dst_hbm, out_hbm,
                  idx_s, idx_d, rows_s0, rows_d0, rows_s1, rows_d1,
                  outbuf, z_sp, sem0, sem1):
        wid = lax.axis_index("s") * NC + lax.axis_index("c")
        sid = lax.axis_index("s")

        @pl.when(sid == 0)
        def _():
            pltpu.sync_copy(z_hbm, z_sp)

        pltpu.sync_copy(src_hbm.at[wid], idx_s)
        pltpu.sync_copy(dst_hbm.at[wid], idx_d)
        plsc.subcore_barrier()

        lanes = lax.broadcasted_iota(jnp.int32, (16,), 0)
        dnums = lax.GatherDimensionNumbers(
            offset_dims=(), collapsed_slice_dims=(0,), start_index_map=(0,))

        def lane_shuffle(v, perm):
            return lax.gather(
                v, perm[:, None], dnums, slice_sizes=(1,),
                mode=lax.GatherScatterMode.PROMISE_IN_BOUNDS)

        def start(c, rs, rd, sem):
            pltpu.async_copy(z_sp.at[idx_s.at[c]], rs, sem)
            pltpu.async_copy(z_sp.at[idx_d.at[c]], rd, sem)

        def drain(c, rs, rd, sem):
            pltpu.make_async_copy(z_sp.at[idx_s.at[c]], rs, sem).wait()
            pltpu.make_async_copy(z_sp.at[idx_d.at[c]], rd, sem).wait()

        def compute(c, rs, rd):
            def group_body(g, carry2):
                vs = []
                for j in range(16):
                    e = g * 16 + j
                    accb = rs[e, pl.ds(0, 32)] * rd[e, pl.ds(0, 32)]
                    for k in range(1, D_FEAT // 32):
                        accb = accb + (rs[e, pl.ds(k * 32, 32)]
                                       * rd[e, pl.ds(k * 32, 32)])
                    a0, a1 = plsc.unpack(
                        accb, format=plsc.PackFormat.INTERLEAVED)
                    vs.append(a0 + a1)
                for k in range(4):
                    step = 1 << k
                    perm = lanes ^ step
                    m = (lanes & step) == 0
                    vs = [jnp.where(m,
                                    a + lane_shuffle(a, perm),
                                    b + lane_shuffle(b, perm))
                          for a, b in zip(vs[0::2], vs[1::2])]
                outbuf[c, pl.ds(g * 16, 16)] = vs[0]
                return carry2

            lax.fori_loop(0, CHUNK // 16, group_body, 0)

        start(0, rows_s0, rows_d0, sem0)

        def pair_body(cc, carry):
            c0 = 2 * cc
            start(c0 + 1, rows_s1, rows_d1, sem1)
            drain(c0, rows_s0, rows_d0, sem0)
            compute(c0, rows_s0, rows_d0)
            start(c0 + 2, rows_s0, rows_d0, sem0)
            drain(c0 + 1, rows_s1, rows_d1, sem1)
            compute(c0 + 1, rows_s1, rows_d1)
            return carry

        lax.fori_loop(0, (NCHUNK - 2) // 2, pair_body, 0)
        c0 = NCHUNK - 2
        start(c0 + 1, rows_s1, rows_d1, sem1)
        drain(c0, rows_s0, rows_d0, sem0)
        compute(c0, rows_s0, rows_d0)
        drain(c0 + 1, rows_s1, rows_d1, sem1)
        compute(c0 + 1, rows_s1, rows_d1)
        pltpu.sync_copy(outbuf, out_hbm.at[wid])

    return sc_logits


def _tc_loss_body(l_ref, o_ref):
    l = l_ref[...]
    rows = lax.broadcasted_iota(jnp.int32, l.shape, 0)
    cols = lax.broadcasted_iota(jnp.int32, l.shape, 1)
    x = jnp.where(rows < (NW * NCHUNK) // 2, -l, l)
    sp = jnp.maximum(x, 0.0) + jnp.log1p(jnp.exp(-jnp.abs(x)))
    sp = jnp.where(cols < CREAL, sp, 0.0)
    o_ref[0, 0] = jnp.sum(sp) * (1.0 / E_TOTAL)


_tc_loss = pl.pallas_call(
    _tc_loss_body,
    out_shape=jax.ShapeDtypeStruct((1, 1), jnp.float32),
    out_specs=pl.BlockSpec(memory_space=pltpu.SMEM),
)


@jax.jit
def kernel(z, pos_edge_index, neg_edge_index):
    total = jnp.concatenate([pos_edge_index, neg_edge_index], axis=1)
    src = total[0].reshape(NW, NCHUNK, CREAL)
    dst = total[1].reshape(NW, NCHUNK, CREAL)
    src = jnp.concatenate([src, src[..., :CHUNK - CREAL]], axis=-1)
    dst = jnp.concatenate([dst, dst[..., :CHUNK - CREAL]], axis=-1)
    logits = _make_sc_logits()(z.astype(jnp.bfloat16), src, dst)
    loss = _tc_loss(logits.reshape(NW * NCHUNK, CHUNK))
    return loss[0, 0]


# speedup vs baseline: 1.0238x; 1.0238x over previous
import functools

import jax
import jax.numpy as jnp
from jax import lax
from jax.experimental import pallas as pl
from jax.experimental.pallas import tpu as pltpu
from jax.experimental.pallas import tpu_sc as plsc

N_NODES = 10000
D_FEAT = 128
E_POS = 160000
E_TOTAL = 320000

NC = 2
NS = 16
NW = NC * NS
EPW = E_TOTAL // NW
CREAL = 125
CHUNK = 128
NCHUNK = EPW // CREAL


def _make_sc_logits():
    mesh = plsc.VectorSubcoreMesh(
        core_axis_name="c", subcore_axis_name="s", num_cores=NC, num_subcores=NS
    )

    @functools.partial(
        pl.kernel,
        mesh=mesh,
        compiler_params=pltpu.CompilerParams(
            needs_layout_passes=False, use_tc_tiling_on_sc=False),
        out_type=jax.ShapeDtypeStruct((NW, NCHUNK, CHUNK), jnp.float32),
        scratch_types=[
            pltpu.VMEM((NCHUNK, CREAL), jnp.int32),
            pltpu.VMEM((NCHUNK, CREAL), jnp.int32),
            pltpu.VMEM((CHUNK, D_FEAT), jnp.bfloat16),
            pltpu.VMEM((CHUNK, D_FEAT), jnp.bfloat16),
            pltpu.VMEM((CHUNK, D_FEAT), jnp.bfloat16),
            pltpu.VMEM((CHUNK, D_FEAT), jnp.bfloat16),
            pltpu.VMEM((NCHUNK, CHUNK), jnp.float32),
            pltpu.VMEM_SHARED((N_NODES, D_FEAT), jnp.bfloat16),
            pltpu.SemaphoreType.DMA,
            pltpu.SemaphoreType.DMA,
        ],
    )
    def sc_logits(z_hbm, pos_s_hbm, pos_d_hbm, neg_s_hbm, neg_d_hbm,
                  out_hbm,
                  idx_s, idx_d, rows_s0, rows_d0, rows_s1, rows_d1,
                  outbuf, z_sp, sem0, sem1):
        wid = lax.axis_index("s") * NC + lax.axis_index("c")
        sid = lax.axis_index("s")

        @pl.when(sid == 0)
        def _():
            pltpu.sync_copy(z_hbm, z_sp)

        @pl.when(wid < NW // 2)
        def _():
            pltpu.sync_copy(pos_s_hbm.at[wid], idx_s)
            pltpu.sync_copy(pos_d_hbm.at[wid], idx_d)

        @pl.when(wid >= NW // 2)
        def _():
            pltpu.sync_copy(neg_s_hbm.at[wid - NW // 2], idx_s)
            pltpu.sync_copy(neg_d_hbm.at[wid - NW // 2], idx_d)

        plsc.subcore_barrier()

        lanes = lax.broadcasted_iota(jnp.int32, (16,), 0)

        def start(c, rs, rd, sem):
            pltpu.async_copy(z_sp.at[idx_s.at[c]], rs.at[pl.ds(0, CREAL)], sem)
            pltpu.async_copy(z_sp.at[idx_d.at[c]], rd.at[pl.ds(0, CREAL)], sem)

        def drain(c, rs, rd, sem):
            pltpu.make_async_copy(
                z_sp.at[idx_s.at[c]], rs.at[pl.ds(0, CREAL)], sem).wait()
            pltpu.make_async_copy(
                z_sp.at[idx_d.at[c]], rd.at[pl.ds(0, CREAL)], sem).wait()

        def compute(c, rs, rd):
            def group_body(g, carry2):
                lvec = jnp.zeros((16,), jnp.float32)
                for j in range(16):
                    e = g * 16 + j
                    accb = rs[e, pl.ds(0, 32)] * rd[e, pl.ds(0, 32)]
                    for k in range(1, D_FEAT // 32):
                        accb = accb + (rs[e, pl.ds(k * 32, 32)]
                                       * rd[e, pl.ds(k * 32, 32)])
                    a0, a1 = plsc.unpack(
                        accb, format=plsc.PackFormat.INTERLEAVED)
                    acc = jnp.sum(a0 + a1)
                    lvec = jnp.where(lanes == j, acc, lvec)
                outbuf[c, pl.ds(g * 16, 16)] = lvec
                return carry2

            lax.fori_loop(0, CHUNK // 16, group_body, 0)

        start(0, rows_s0, rows_d0, sem0)

        def pair_body(cc, carry):
            c0 = 2 * cc
            start(c0 + 1, rows_s1, rows_d1, sem1)
            drain(c0, rows_s0, rows_d0, sem0)
            compute(c0, rows_s0, rows_d0)
            start(c0 + 2, rows_s0, rows_d0, sem0)
            drain(c0 + 1, rows_s1, rows_d1, sem1)
            compute(c0 + 1, rows_s1, rows_d1)
            return carry

        lax.fori_loop(0, (NCHUNK - 2) // 2, pair_body, 0)
        c0 = NCHUNK - 2
        start(c0 + 1, rows_s1, rows_d1, sem1)
        drain(c0, rows_s0, rows_d0, sem0)
        compute(c0, rows_s0, rows_d0)
        drain(c0 + 1, rows_s1, rows_d1, sem1)
        compute(c0 + 1, rows_s1, rows_d1)
        pltpu.sync_copy(outbuf, out_hbm.at[wid])

    return sc_logits


def _tc_loss_body(l_ref, o_ref):
    l = l_ref[...]
    rows = lax.broadcasted_iota(jnp.int32, l.shape, 0)
    cols = lax.broadcasted_iota(jnp.int32, l.shape, 1)
    x = jnp.where(rows < (NW * NCHUNK) // 2, -l, l)
    sp = jnp.maximum(x, 0.0) + jnp.log1p(jnp.exp(-jnp.abs(x)))
    sp = jnp.where(cols < CREAL, sp, 0.0)
    o_ref[0, 0] = jnp.sum(sp) * (1.0 / E_TOTAL)


_tc_loss = pl.pallas_call(
    _tc_loss_body,
    out_shape=jax.ShapeDtypeStruct((1, 1), jnp.float32),
    out_specs=pl.BlockSpec(memory_space=pltpu.SMEM),
)


@jax.jit
def kernel(z, pos_edge_index, neg_edge_index):
    h = NW // 2
    ps = pos_edge_index[0].reshape(h, NCHUNK, CREAL)
    pd = pos_edge_index[1].reshape(h, NCHUNK, CREAL)
    ns = neg_edge_index[0].reshape(h, NCHUNK, CREAL)
    nd = neg_edge_index[1].reshape(h, NCHUNK, CREAL)
    logits = _make_sc_logits()(z.astype(jnp.bfloat16), ps, pd, ns, nd)
    loss = _tc_loss(logits.reshape(NW * NCHUNK, CHUNK))
    return loss[0, 0]


# speedup vs baseline: 1.0372x; 1.0131x over previous
import functools

import jax
import jax.numpy as jnp
from jax import lax
from jax.experimental import pallas as pl
from jax.experimental.pallas import tpu as pltpu
from jax.experimental.pallas import tpu_sc as plsc

N_NODES = 10000
D_FEAT = 128
E_POS = 160000
E_TOTAL = 320000

NC = 2
NS = 16
NW = NC * NS
EPW = E_TOTAL // NW
CREAL = 125
CHUNK = 128
NCHUNK = EPW // CREAL


def _make_sc_logits():
    mesh = plsc.VectorSubcoreMesh(
        core_axis_name="c", subcore_axis_name="s", num_cores=NC, num_subcores=NS
    )

    @functools.partial(
        pl.kernel,
        mesh=mesh,
        compiler_params=pltpu.CompilerParams(
            needs_layout_passes=False, use_tc_tiling_on_sc=False),
        out_type=jax.ShapeDtypeStruct((NW, NCHUNK, CHUNK), jnp.float32),
        scratch_types=[
            pltpu.VMEM((NCHUNK, CREAL), jnp.int32),
            pltpu.VMEM((NCHUNK, CREAL), jnp.int32),
            pltpu.VMEM((CHUNK, D_FEAT), jnp.bfloat16),
            pltpu.VMEM((CHUNK, D_FEAT), jnp.bfloat16),
            pltpu.VMEM((CHUNK, D_FEAT), jnp.bfloat16),
            pltpu.VMEM((CHUNK, D_FEAT), jnp.bfloat16),
            pltpu.VMEM((NCHUNK, CHUNK), jnp.float32),
            pltpu.VMEM_SHARED((N_NODES, D_FEAT), jnp.bfloat16),
            pltpu.SemaphoreType.DMA,
            pltpu.SemaphoreType.DMA,
            pltpu.SemaphoreType.DMA,
        ],
    )
    def sc_logits(z_hbm, pos_s_hbm, pos_d_hbm, neg_s_hbm, neg_d_hbm,
                  out_hbm,
                  idx_s, idx_d, rows_s0, rows_d0, rows_s1, rows_d1,
                  outbuf, z_sp, sem0, sem1, semz):
        wid = lax.axis_index("s") * NC + lax.axis_index("c")
        sid = lax.axis_index("s")

        zrows = N_NODES // NS
        cpz = pltpu.async_copy(
            z_hbm.at[pl.ds(sid * zrows, zrows)],
            z_sp.at[pl.ds(sid * zrows, zrows)], semz)

        @pl.when(wid < NW // 2)
        def _():
            pltpu.sync_copy(pos_s_hbm.at[wid], idx_s)
            pltpu.sync_copy(pos_d_hbm.at[wid], idx_d)

        @pl.when(wid >= NW // 2)
        def _():
            pltpu.sync_copy(neg_s_hbm.at[wid - NW // 2], idx_s)
            pltpu.sync_copy(neg_d_hbm.at[wid - NW // 2], idx_d)

        cpz.wait()
        plsc.subcore_barrier()

        lanes = lax.broadcasted_iota(jnp.int32, (16,), 0)

        def start(c, rs, rd, sem):
            pltpu.async_copy(z_sp.at[idx_s.at[c]], rs.at[pl.ds(0, CREAL)], sem)
            pltpu.async_copy(z_sp.at[idx_d.at[c]], rd.at[pl.ds(0, CREAL)], sem)

        def drain(c, rs, rd, sem):
            pltpu.make_async_copy(
                z_sp.at[idx_s.at[c]], rs.at[pl.ds(0, CREAL)], sem).wait()
            pltpu.make_async_copy(
                z_sp.at[idx_d.at[c]], rd.at[pl.ds(0, CREAL)], sem).wait()

        def compute(c, rs, rd):
            def group_body(g, carry2):
                lvec = jnp.zeros((16,), jnp.float32)
                for j in range(16):
                    e = g * 16 + j
                    accb = rs[e, pl.ds(0, 32)] * rd[e, pl.ds(0, 32)]
                    for k in range(1, D_FEAT // 32):
                        accb = accb + (rs[e, pl.ds(k * 32, 32)]
                                       * rd[e, pl.ds(k * 32, 32)])
                    a0, a1 = plsc.unpack(
                        accb, format=plsc.PackFormat.INTERLEAVED)
                    acc = jnp.sum(a0 + a1)
                    lvec = jnp.where(lanes == j, acc, lvec)
                outbuf[c, pl.ds(g * 16, 16)] = lvec
                return carry2

            lax.fori_loop(0, CHUNK // 16, group_body, 0)

        start(0, rows_s0, rows_d0, sem0)

        def pair_body(cc, carry):
            c0 = 2 * cc
            start(c0 + 1, rows_s1, rows_d1, sem1)
            drain(c0, rows_s0, rows_d0, sem0)
            compute(c0, rows_s0, rows_d0)
            start(c0 + 2, rows_s0, rows_d0, sem0)
            drain(c0 + 1, rows_s1, rows_d1, sem1)
            compute(c0 + 1, rows_s1, rows_d1)
            return carry

        lax.fori_loop(0, (NCHUNK - 2) // 2, pair_body, 0)
        c0 = NCHUNK - 2
        start(c0 + 1, rows_s1, rows_d1, sem1)
        drain(c0, rows_s0, rows_d0, sem0)
        compute(c0, rows_s0, rows_d0)
        drain(c0 + 1, rows_s1, rows_d1, sem1)
        compute(c0 + 1, rows_s1, rows_d1)
        pltpu.sync_copy(outbuf, out_hbm.at[wid])

    return sc_logits


def _tc_loss_body(l_ref, o_ref):
    l = l_ref[...]
    rows = lax.broadcasted_iota(jnp.int32, l.shape, 0)
    cols = lax.broadcasted_iota(jnp.int32, l.shape, 1)
    x = jnp.where(rows < (NW * NCHUNK) // 2, -l, l)
    sp = jnp.maximum(x, 0.0) + jnp.log1p(jnp.exp(-jnp.abs(x)))
    sp = jnp.where(cols < CREAL, sp, 0.0)
    o_ref[0, 0] = jnp.sum(sp) * (1.0 / E_TOTAL)


_tc_loss = pl.pallas_call(
    _tc_loss_body,
    out_shape=jax.ShapeDtypeStruct((1, 1), jnp.float32),
    out_specs=pl.BlockSpec(memory_space=pltpu.SMEM),
)


@jax.jit
def kernel(z, pos_edge_index, neg_edge_index):
    h = NW // 2
    ps = pos_edge_index[0].reshape(h, NCHUNK, CREAL)
    pd = pos_edge_index[1].reshape(h, NCHUNK, CREAL)
    ns = neg_edge_index[0].reshape(h, NCHUNK, CREAL)
    nd = neg_edge_index[1].reshape(h, NCHUNK, CREAL)
    logits = _make_sc_logits()(z.astype(jnp.bfloat16), ps, pd, ns, nd)
    loss = _tc_loss(logits.reshape(NW * NCHUNK, CHUNK))
    return loss[0, 0]
